# baseline (speedup 1.0000x reference)
import functools

import jax
import jax.numpy as jnp
from jax.experimental import pallas as pl
from jax.experimental.pallas import tpu as pltpu


def _mlp_kernel(e_ref, w1t_ref, b1_ref, w2_ref, out_ref):
    h = jnp.dot(e_ref[...], w1t_ref[...], preferred_element_type=jnp.float32)
    h = jnp.maximum(h + b1_ref[...], 0.0)
    out_ref[...] = jnp.sum(h * w2_ref[...], axis=1, keepdims=True).astype(out_ref.dtype)


def _round_up(n, m):
    return ((n + m - 1) // m) * m


@jax.jit
def _forward(x, W, H, w1, b1, w2):
    B = x.shape[0]
    K = W.shape[1]

    user_idx = x[:, 0].astype(jnp.int32)
    item_idx = x[:, 1].astype(jnp.int32)

    e = jnp.concatenate(
        [jnp.take(W, user_idx, axis=0), jnp.take(H, item_idx, axis=0)], axis=1)

    TB = min(4096, _round_up(B, 256) // 2)
    TB = max(256, (TB // 256) * 256)
    B_pad = _round_up(B, TB)
    if B_pad != B:
        e = jnp.pad(e, ((0, B_pad - B), (0, 0)))

    out = pl.pallas_call(
        _mlp_kernel,
        out_shape=jax.ShapeDtypeStruct((B_pad, 1), jnp.float32),
        grid=(B_pad // TB,),
        in_specs=[
            pl.BlockSpec((TB, 2 * K), lambda i: (i, 0)),
            pl.BlockSpec((2 * K, K), lambda i: (0, 0)),
            pl.BlockSpec((1, K), lambda i: (0, 0)),
            pl.BlockSpec((1, K), lambda i: (0, 0)),
        ],
        out_specs=pl.BlockSpec((TB, 1), lambda i: (i, 0)),
        compiler_params=pltpu.CompilerParams(
            dimension_semantics=("parallel",),
        ),
    )(e, w1.T, b1.reshape(1, K), w2.reshape(1, K))

    return out[:B]


def kernel(x, W, H, w1, b1, w2):
    return _forward(x, W, H, w1, b1, w2)


# speedup vs baseline: 1.1229x; 1.1229x over previous
import functools

import jax
import jax.numpy as jnp
from jax.experimental import pallas as pl
from jax.experimental.pallas import tpu as pltpu


def _mlp_kernel(eu_ref, ev_ref, w1at_ref, w1bt_ref, b1_ref, w2_ref, out_ref):
    h = (jnp.dot(eu_ref[...], w1at_ref[...], preferred_element_type=jnp.float32)
         + jnp.dot(ev_ref[...], w1bt_ref[...], preferred_element_type=jnp.float32))
    h = jnp.maximum(h + b1_ref[...], 0.0)
    out_ref[...] = jnp.sum(h * w2_ref[...], axis=1, keepdims=True).astype(out_ref.dtype)


def _round_up(n, m):
    return ((n + m - 1) // m) * m


@jax.jit
def _forward(x, W, H, w1, b1, w2):
    B = x.shape[0]
    K = W.shape[1]

    user_idx = x[:, 0].astype(jnp.int32)
    item_idx = x[:, 1].astype(jnp.int32)

    eu = jnp.take(W, user_idx, axis=0)
    ev = jnp.take(H, item_idx, axis=0)

    TB = min(4096, _round_up(B, 256) // 2)
    TB = max(256, (TB // 256) * 256)
    B_pad = _round_up(B, TB)
    if B_pad != B:
        eu = jnp.pad(eu, ((0, B_pad - B), (0, 0)))
        ev = jnp.pad(ev, ((0, B_pad - B), (0, 0)))

    out = pl.pallas_call(
        _mlp_kernel,
        out_shape=jax.ShapeDtypeStruct((B_pad, 1), jnp.float32),
        grid=(B_pad // TB,),
        in_specs=[
            pl.BlockSpec((TB, K), lambda i: (i, 0)),
            pl.BlockSpec((TB, K), lambda i: (i, 0)),
            pl.BlockSpec((K, K), lambda i: (0, 0)),
            pl.BlockSpec((K, K), lambda i: (0, 0)),
            pl.BlockSpec((1, K), lambda i: (0, 0)),
            pl.BlockSpec((1, K), lambda i: (0, 0)),
        ],
        out_specs=pl.BlockSpec((TB, 1), lambda i: (i, 0)),
        compiler_params=pltpu.CompilerParams(
            dimension_semantics=("parallel",),
        ),
    )(eu, ev, w1[:, :K].T, w1[:, K:].T, b1.reshape(1, K), w2.reshape(1, K))

    return out[:B]


def kernel(x, W, H, w1, b1, w2):
    return _forward(x, W, H, w1, b1, w2)


# speedup vs baseline: 1.3924x; 1.2400x over previous
import functools

import jax
import jax.numpy as jnp
from jax.experimental import pallas as pl
from jax.experimental.pallas import tpu as pltpu


def _mlp_kernel(eu_ref, ev_ref, w1at_ref, w1bt_ref, b1_ref, w2_ref, out_ref):
    h = (jnp.dot(eu_ref[...], w1at_ref[...], preferred_element_type=jnp.float32)
         + jnp.dot(ev_ref[...], w1bt_ref[...], preferred_element_type=jnp.float32))
    h = jnp.maximum(h + b1_ref[...], 0.0)
    out_ref[...] = jnp.dot(h, w2_ref[...],
                           preferred_element_type=jnp.float32).astype(out_ref.dtype)


def _round_up(n, m):
    return ((n + m - 1) // m) * m


@jax.jit
def _forward(x, W, H, w1, b1, w2):
    B = x.shape[0]
    K = W.shape[1]

    user_idx = x[:, 0].astype(jnp.int32)
    item_idx = x[:, 1].astype(jnp.int32)

    eu = W.at[user_idx].get(mode="promise_in_bounds")
    ev = H.at[item_idx].get(mode="promise_in_bounds")

    TB = min(8192, _round_up(B, 256) // 2)
    TB = max(256, (TB // 256) * 256)
    B_pad = _round_up(B, TB)
    if B_pad != B:
        eu = jnp.pad(eu, ((0, B_pad - B), (0, 0)))
        ev = jnp.pad(ev, ((0, B_pad - B), (0, 0)))

    out = pl.pallas_call(
        _mlp_kernel,
        out_shape=jax.ShapeDtypeStruct((B_pad, 1), jnp.float32),
        grid=(B_pad // TB,),
        in_specs=[
            pl.BlockSpec((TB, K), lambda i: (i, 0)),
            pl.BlockSpec((TB, K), lambda i: (i, 0)),
            pl.BlockSpec((K, K), lambda i: (0, 0)),
            pl.BlockSpec((K, K), lambda i: (0, 0)),
            pl.BlockSpec((1, K), lambda i: (0, 0)),
            pl.BlockSpec((K, 1), lambda i: (0, 0)),
        ],
        out_specs=pl.BlockSpec((TB, 1), lambda i: (i, 0)),
        compiler_params=pltpu.CompilerParams(
            dimension_semantics=("parallel",),
        ),
    )(eu, ev, w1[:, :K].T, w1[:, K:].T, b1.reshape(1, K), w2.reshape(K, 1))

    return out[:B]


def kernel(x, W, H, w1, b1, w2):
    return _forward(x, W, H, w1, b1, w2)
